# Initial kernel scaffold; baseline (speedup 1.0000x reference)
import jax
import jax.numpy as jnp
from jax.experimental import pallas as pl


def kernel(src_org_edge_feat, src_edge_to_time, src_center_node_idx, src_neigh_edge, src_node_features, params):
    raise NotImplementedError("write your pallas kernel here")


# speedup vs baseline: 8.4834x; 8.4834x over previous
import functools

import jax
import jax.numpy as jnp
import numpy as np
from jax.experimental import pallas as pl
from jax.experimental.pallas import tpu as pltpu

D = 128
H = 2
DH = D // H
L = 2


def _edge_body(x_ref, t_ref, w1_ref, b1_ref, w2_ref, b2_ref, wef_ref, wt_ref,
               tf_ref, tp_ref, out_ref):
    x = x_ref[...]
    h1 = jnp.maximum(
        jnp.dot(x, w1_ref[...], preferred_element_type=jnp.float32) + b1_ref[...], 0.0)
    ef = jnp.dot(h1, w2_ref[...], preferred_element_type=jnp.float32) + b2_ref[...]
    phi = jnp.cos(t_ref[...] * tf_ref[...] + tp_ref[...])
    out_ref[...] = (
        jnp.dot(ef, wef_ref[...], preferred_element_type=jnp.float32)
        + jnp.dot(phi, wt_ref[...], preferred_element_type=jnp.float32))


def _edge_kv(edge_feat, t, w1, b1, w2, b2, wef, wt, tfreq, tphase, blk=512):
    E = edge_feat.shape[0]
    nout = wef.shape[1]
    grid = (E // blk,)
    return pl.pallas_call(
        _edge_body,
        grid=grid,
        in_specs=[
            pl.BlockSpec((blk, D), lambda i: (i, 0)),
            pl.BlockSpec((blk, 1), lambda i: (i, 0)),
            pl.BlockSpec((D, D), lambda i: (0, 0)),
            pl.BlockSpec((1, D), lambda i: (0, 0)),
            pl.BlockSpec((D, D), lambda i: (0, 0)),
            pl.BlockSpec((1, D), lambda i: (0, 0)),
            pl.BlockSpec((D, nout), lambda i: (0, 0)),
            pl.BlockSpec((D, nout), lambda i: (0, 0)),
            pl.BlockSpec((1, D), lambda i: (0, 0)),
            pl.BlockSpec((1, D), lambda i: (0, 0)),
        ],
        out_specs=pl.BlockSpec((blk, nout), lambda i: (i, 0)),
        out_shape=jax.ShapeDtypeStruct((E, nout), jnp.float32),
    )(edge_feat, t, w1, b1, w2, b2, wef, wt, tfreq, tphase)


def _mlp2_body(x_ref, w1_ref, b1_ref, w2_ref, b2_ref, out_ref):
    h1 = jnp.maximum(
        jnp.dot(x_ref[...], w1_ref[...], preferred_element_type=jnp.float32)
        + b1_ref[...], 0.0)
    out_ref[...] = (jnp.dot(h1, w2_ref[...], preferred_element_type=jnp.float32)
                    + b2_ref[...])


def _mlp2(x, w1, b1, w2, b2, blk=512):
    M, K = x.shape
    K2, N2 = w2.shape
    grid = (pl.cdiv(M, blk),)
    return pl.pallas_call(
        _mlp2_body,
        grid=grid,
        in_specs=[
            pl.BlockSpec((blk, K), lambda i: (i, 0)),
            pl.BlockSpec((K, K2), lambda i: (0, 0)),
            pl.BlockSpec((1, K2), lambda i: (0, 0)),
            pl.BlockSpec((K2, N2), lambda i: (0, 0)),
            pl.BlockSpec((1, N2), lambda i: (0, 0)),
        ],
        out_specs=pl.BlockSpec((blk, N2), lambda i: (i, 0)),
        out_shape=jax.ShapeDtypeStruct((M, N2), jnp.float32),
    )(x, w1, b1, w2, b2)


def _proj_body(h_ref, w_ref, b_ref, out_ref):
    out_ref[...] = (jnp.dot(h_ref[...], w_ref[...], preferred_element_type=jnp.float32)
                    + b_ref[...])


def _proj(h, w, b, blk=512):
    M, K = h.shape
    N2 = w.shape[1]
    grid = (pl.cdiv(M, blk),)
    return pl.pallas_call(
        _proj_body,
        grid=grid,
        in_specs=[
            pl.BlockSpec((blk, K), lambda i: (i, 0)),
            pl.BlockSpec((K, N2), lambda i: (0, 0)),
            pl.BlockSpec((1, N2), lambda i: (0, 0)),
        ],
        out_specs=pl.BlockSpec((blk, N2), lambda i: (i, 0)),
        out_shape=jax.ShapeDtypeStruct((M, N2), jnp.float32),
    )(h, w, b)


def _upd_body(agg_ref, h_ref, w1a_ref, w1b_ref, b1_ref, w2_ref, b2_ref, out_ref):
    z = (jnp.dot(agg_ref[...], w1a_ref[...], preferred_element_type=jnp.float32)
         + jnp.dot(h_ref[...], w1b_ref[...], preferred_element_type=jnp.float32)
         + b1_ref[...])
    z = jnp.maximum(z, 0.0)
    out_ref[...] = (jnp.dot(z, w2_ref[...], preferred_element_type=jnp.float32)
                    + b2_ref[...])


def _upd(agg, h, w1a, w1b, b1, w2, b2, blk=512):
    M = h.shape[0]
    grid = (pl.cdiv(M, blk),)
    return pl.pallas_call(
        _upd_body,
        grid=grid,
        in_specs=[
            pl.BlockSpec((blk, D), lambda i: (i, 0)),
            pl.BlockSpec((blk, D), lambda i: (i, 0)),
            pl.BlockSpec((D, D), lambda i: (0, 0)),
            pl.BlockSpec((D, D), lambda i: (0, 0)),
            pl.BlockSpec((1, D), lambda i: (0, 0)),
            pl.BlockSpec((D, D), lambda i: (0, 0)),
            pl.BlockSpec((1, D), lambda i: (0, 0)),
        ],
        out_specs=pl.BlockSpec((blk, D), lambda i: (i, 0)),
        out_shape=jax.ShapeDtypeStruct((M, D), jnp.float32),
    )(agg, h, w1a, w1b, b1, w2, b2)


def kernel(src_org_edge_feat, src_edge_to_time, src_center_node_idx,
           src_neigh_edge, src_node_features, params):
    p = params
    E = src_neigh_edge.shape[0]
    N = src_node_features.shape[0]
    dst = src_neigh_edge[:, 0]
    src = src_neigh_edge[:, 1]

    phi0 = jnp.cos(p["time_phase"])
    wef = jnp.concatenate([p[f"W{kv}{l}"][D:2 * D] for l in range(L)
                           for kv in ("k", "v")], axis=1)
    wt = jnp.concatenate([p[f"W{kv}{l}"][2 * D:3 * D] for l in range(L)
                          for kv in ("k", "v")], axis=1)
    projs = []
    for l in range(L):
        w = jnp.concatenate([p[f"Wq{l}"][:D], p[f"Wk{l}"][:D], p[f"Wv{l}"][:D]],
                            axis=1)
        q0 = phi0 @ p[f"Wq{l}"][D:]
        b = jnp.concatenate([q0, jnp.zeros((2 * D,), jnp.float32)])[None, :]
        projs.append((w, b))
    wa1 = p["Wa1"][:D] + p["Wa1"][D:]

    t2 = src_edge_to_time[:, None]
    tf2 = p["time_freq"][None, :]
    tp2 = p["time_phase"][None, :]

    kv_all = _edge_kv(src_org_edge_feat, t2, p["edge_W1"], p["edge_b1"][None, :],
                      p["edge_W2"], p["edge_b2"][None, :], wef, wt, tf2, tp2)
    h = _mlp2(src_node_features, p["node_W1"], p["node_b1"][None, :],
              p["node_W2"], p["node_b2"][None, :])

    inv_sqrt = np.float32(1.0 / np.sqrt(DH))
    for l in range(L):
        qkvn = _proj(h, projs[l][0], projs[l][1])
        qn, kn, vn = qkvn[:, :D], qkvn[:, D:2 * D], qkvn[:, 2 * D:]
        ke = kv_all[:, (2 * l) * D:(2 * l + 1) * D]
        ve = kv_all[:, (2 * l + 1) * D:(2 * l + 2) * D]
        q = qn[dst]
        k = kn[src] + ke
        v = vn[src] + ve
        s = jnp.sum(q.reshape(E, H, DH) * k.reshape(E, H, DH), axis=-1) * inv_sqrt
        ex = jnp.exp(s)
        den = jax.ops.segment_sum(ex, dst, num_segments=N)
        num = jax.ops.segment_sum(
            (ex[:, :, None] * v.reshape(E, H, DH)).reshape(E, D), dst,
            num_segments=N)
        agg = (num.reshape(N, H, DH)
               / (den[:, :, None] + 1e-10)).reshape(N, D)
        h = _upd(agg, h, p[f"Wm1_{l}"][:D], p[f"Wm1_{l}"][D:],
                 p[f"bm1_{l}"][None, :], p[f"Wm2_{l}"], p[f"bm2_{l}"][None, :])

    x = h[src_center_node_idx]
    score = _mlp2(x, wa1, p["ba1"][None, :], p["Wa2"],
                  p["ba2"][None, :], blk=1024)
    return score


# speedup vs baseline: 12.6320x; 1.4890x over previous
import functools

import jax
import jax.numpy as jnp
import numpy as np
from jax import lax
from jax.experimental import pallas as pl
from jax.experimental.pallas import tpu as pltpu
from jax.experimental.pallas import tpu_sc as plsc

D = 128
H = 2
DH = D // H
L = 2

_NC = 2
_NS = 16
_LANES = 16


def _edge_body(x_ref, t_ref, w1_ref, b1_ref, w2_ref, b2_ref, wef_ref, wt_ref,
               tf_ref, tp_ref, kv0_ref, kv1_ref):
    x = x_ref[...]
    h1 = jnp.maximum(
        jnp.dot(x, w1_ref[...], preferred_element_type=jnp.float32) + b1_ref[...], 0.0)
    ef = jnp.dot(h1, w2_ref[...], preferred_element_type=jnp.float32) + b2_ref[...]
    phi = jnp.cos(t_ref[...] * tf_ref[...] + tp_ref[...])
    out = (jnp.dot(ef, wef_ref[...], preferred_element_type=jnp.float32)
           + jnp.dot(phi, wt_ref[...], preferred_element_type=jnp.float32))
    kv0_ref[...] = out[:, :2 * D]
    kv1_ref[...] = out[:, 2 * D:]


def _edge_kv(edge_feat, t, w1, b1, w2, b2, wef, wt, tfreq, tphase, blk=512):
    E = edge_feat.shape[0]
    nout = wef.shape[1]
    grid = (E // blk,)
    return pl.pallas_call(
        _edge_body,
        grid=grid,
        in_specs=[
            pl.BlockSpec((blk, D), lambda i: (i, 0)),
            pl.BlockSpec((blk, 1), lambda i: (i, 0)),
            pl.BlockSpec((D, D), lambda i: (0, 0)),
            pl.BlockSpec((1, D), lambda i: (0, 0)),
            pl.BlockSpec((D, D), lambda i: (0, 0)),
            pl.BlockSpec((1, D), lambda i: (0, 0)),
            pl.BlockSpec((D, nout), lambda i: (0, 0)),
            pl.BlockSpec((D, nout), lambda i: (0, 0)),
            pl.BlockSpec((1, D), lambda i: (0, 0)),
            pl.BlockSpec((1, D), lambda i: (0, 0)),
        ],
        out_specs=[pl.BlockSpec((blk, 2 * D), lambda i: (i, 0)),
                   pl.BlockSpec((blk, 2 * D), lambda i: (i, 0))],
        out_shape=[jax.ShapeDtypeStruct((E, 2 * D), jnp.float32),
                   jax.ShapeDtypeStruct((E, 2 * D), jnp.float32)],
    )(edge_feat, t, w1, b1, w2, b2, wef, wt, tfreq, tphase)


def _mlp2_body(x_ref, w1_ref, b1_ref, w2_ref, b2_ref, out_ref):
    h1 = jnp.maximum(
        jnp.dot(x_ref[...], w1_ref[...], preferred_element_type=jnp.float32)
        + b1_ref[...], 0.0)
    out_ref[...] = (jnp.dot(h1, w2_ref[...], preferred_element_type=jnp.float32)
                    + b2_ref[...])


def _mlp2(x, w1, b1, w2, b2, blk=512):
    M, K = x.shape
    K2, N2 = w2.shape
    grid = (pl.cdiv(M, blk),)
    return pl.pallas_call(
        _mlp2_body,
        grid=grid,
        in_specs=[
            pl.BlockSpec((blk, K), lambda i: (i, 0)),
            pl.BlockSpec((K, K2), lambda i: (0, 0)),
            pl.BlockSpec((1, K2), lambda i: (0, 0)),
            pl.BlockSpec((K2, N2), lambda i: (0, 0)),
            pl.BlockSpec((1, N2), lambda i: (0, 0)),
        ],
        out_specs=pl.BlockSpec((blk, N2), lambda i: (i, 0)),
        out_shape=jax.ShapeDtypeStruct((M, N2), jnp.float32),
    )(x, w1, b1, w2, b2)


def _proj_body(h_ref, w_ref, b_ref, out_ref):
    out_ref[...] = (jnp.dot(h_ref[...], w_ref[...], preferred_element_type=jnp.float32)
                    + b_ref[...])


def _proj(h, w, b, blk=512):
    M, K = h.shape
    N2 = w.shape[1]
    grid = (pl.cdiv(M, blk),)
    return pl.pallas_call(
        _proj_body,
        grid=grid,
        in_specs=[
            pl.BlockSpec((blk, K), lambda i: (i, 0)),
            pl.BlockSpec((K, N2), lambda i: (0, 0)),
            pl.BlockSpec((1, N2), lambda i: (0, 0)),
        ],
        out_specs=pl.BlockSpec((blk, N2), lambda i: (i, 0)),
        out_shape=jax.ShapeDtypeStruct((M, N2), jnp.float32),
    )(h, w, b)


def _upd_body(blk, acc0_ref, acc1_ref, den0_ref, den1_ref, h_ref,
              w1a_ref, w1b_ref, b1_ref, w2_ref, b2_ref, out_ref):
    den = den0_ref[0] + den1_ref[0]
    div = jnp.concatenate(
        [jnp.broadcast_to(den[:, 0:1], (blk, DH)),
         jnp.broadcast_to(den[:, 1:2], (blk, DH))], axis=1) + 1e-10
    agg = (acc0_ref[0] + acc1_ref[0]) / div
    z = (jnp.dot(agg, w1a_ref[...], preferred_element_type=jnp.float32)
         + jnp.dot(h_ref[...], w1b_ref[...], preferred_element_type=jnp.float32)
         + b1_ref[...])
    z = jnp.maximum(z, 0.0)
    out_ref[...] = (jnp.dot(z, w2_ref[...], preferred_element_type=jnp.float32)
                    + b2_ref[...])


def _upd(accp, denp, h, w1a, w1b, b1, w2, b2, blk=512):
    M = h.shape[0]
    grid = (pl.cdiv(M, blk),)
    return pl.pallas_call(
        functools.partial(_upd_body, blk),
        grid=grid,
        in_specs=[
            pl.BlockSpec((1, blk, D), lambda i: (0, i, 0)),
            pl.BlockSpec((1, blk, D), lambda i: (1, i, 0)),
            pl.BlockSpec((1, blk, D), lambda i: (0, i, 0)),
            pl.BlockSpec((1, blk, D), lambda i: (1, i, 0)),
            pl.BlockSpec((blk, D), lambda i: (i, 0)),
            pl.BlockSpec((D, D), lambda i: (0, 0)),
            pl.BlockSpec((D, D), lambda i: (0, 0)),
            pl.BlockSpec((1, D), lambda i: (0, 0)),
            pl.BlockSpec((D, D), lambda i: (0, 0)),
            pl.BlockSpec((1, D), lambda i: (0, 0)),
        ],
        out_specs=pl.BlockSpec((blk, D), lambda i: (i, 0)),
        out_shape=jax.ShapeDtypeStruct((M, D), jnp.float32),
    )(accp, accp, denp, denp, h, w1a, w1b, b1, w2, b2)


def _sc_edge_pass(dst3, src3, qn, kn, vn, kv):
    NW, NCH3, C3 = dst3.shape
    E = NW * NCH3 * C3
    N = qn.shape[0]
    EPW = E // NW
    C = 40
    NCH = EPW // C
    assert EPW * NW == E and NCH * C == EPW
    RPT = (N // _NS) // 8 * 8
    TAIL = N - RPT * _NS
    NFULL, REM = RPT // C, RPT % C
    assert REM % 8 == 0 and TAIL <= C and TAIL % 8 == 0
    scale = np.float32(1.0 / np.sqrt(DH))

    mesh = plsc.VectorSubcoreMesh(core_axis_name="c", subcore_axis_name="s",
                                  num_cores=_NC, num_subcores=_NS)

    @functools.partial(
        pl.kernel,
        out_type=[jax.ShapeDtypeStruct((_NC, N, D), jnp.float32),
                  jax.ShapeDtypeStruct((E, _LANES), jnp.float32)],
        mesh=mesh,
        scratch_types=[
            pltpu.VMEM((C,), jnp.int32),
            pltpu.VMEM((C,), jnp.int32),
            pltpu.VMEM((C, D), jnp.float32),
            pltpu.VMEM((C, D), jnp.float32),
            pltpu.VMEM((C, D), jnp.float32),
            pltpu.VMEM((C, 2 * D), jnp.float32),
            pltpu.VMEM((C, D), jnp.float32),
            pltpu.VMEM((C, _LANES), jnp.float32),
            pltpu.VMEM_SHARED((N, D), jnp.float32),
            pltpu.SemaphoreType.DMA,
        ],
    )
    def sc_kernel(dst_hbm, src_hbm, qn_hbm, kn_hbm, vn_hbm, kv_hbm,
                  acc_hbm, exr_hbm,
                  dst_v, src_v, q_v, k_v, v_v, kv_v, out_v, dden_v,
                  acc_sh, sem):
        cid = lax.axis_index("c")
        sid = lax.axis_index("s")
        wid = sid * _NC + cid

        def zrow(i, _):
            z = jnp.zeros((_LANES,), jnp.float32)
            for j in range(D // _LANES):
                out_v[i, _LANES * j:_LANES * (j + 1)] = z
            return 0
        lax.fori_loop(0, C, zrow, 0)

        def zcopy(m, _):
            pltpu.sync_copy(out_v, acc_sh.at[pl.ds(sid * RPT + m * C, C)])
            return 0
        lax.fori_loop(0, NFULL, zcopy, 0)
        if REM:
            pltpu.sync_copy(out_v.at[pl.ds(0, REM)],
                            acc_sh.at[pl.ds(sid * RPT + NFULL * C, REM)])
        @pl.when(sid == 0)
        def _():
            pltpu.sync_copy(out_v.at[pl.ds(0, TAIL)],
                            acc_sh.at[pl.ds(_NS * RPT, TAIL)])
        plsc.subcore_barrier()

        def edge_body(i, _):
            lane = lax.broadcasted_iota(jnp.int32, (_LANES,), 0)

            def bfly(v):
                for sh in (8, 4, 2, 1):
                    v = v + v.at[lane ^ sh].get(mode="promise_in_bounds")
                return v

            s0 = jnp.zeros((_LANES,), jnp.float32)
            s1 = jnp.zeros((_LANES,), jnp.float32)
            for j in range(4):
                o = _LANES * j
                s0 = s0 + q_v[i, o:o + _LANES] * (
                    k_v[i, o:o + _LANES] + kv_v[i, o:o + _LANES])
            for j in range(4, 8):
                o = _LANES * j
                s1 = s1 + q_v[i, o:o + _LANES] * (
                    k_v[i, o:o + _LANES] + kv_v[i, o:o + _LANES])
            ex0 = jnp.exp(bfly(s0) * scale)
            ex1 = jnp.exp(bfly(s1) * scale)
            for j in range(4):
                o = _LANES * j
                out_v[i, o:o + _LANES] = ex0 * (
                    v_v[i, o:o + _LANES] + kv_v[i, D + o:D + o + _LANES])
            for j in range(4, 8):
                o = _LANES * j
                out_v[i, o:o + _LANES] = ex1 * (
                    v_v[i, o:o + _LANES] + kv_v[i, D + o:D + o + _LANES])
            dden_v[i, :] = jnp.where(lane == 0, ex0,
                                     jnp.where(lane == 1, ex1, 0.0))
            return 0

        def chunk_body(j, _):
            base = wid * EPW + j * C
            pltpu.sync_copy(dst_hbm.at[wid, j], dst_v)
            pltpu.sync_copy(src_hbm.at[wid, j], src_v)
            pltpu.async_copy(qn_hbm.at[dst_v], q_v, sem).wait()
            pltpu.async_copy(kn_hbm.at[src_v], k_v, sem).wait()
            pltpu.async_copy(vn_hbm.at[src_v], v_v, sem).wait()
            pltpu.sync_copy(kv_hbm.at[pl.ds(base, C)], kv_v)
            lax.fori_loop(0, C, edge_body, 0)
            pltpu.sync_copy(out_v, acc_sh.at[dst_v], add=True)
            pltpu.sync_copy(dden_v, exr_hbm.at[pl.ds(base, C)])
            return 0

        lax.fori_loop(0, NCH, chunk_body, 0)
        plsc.subcore_barrier()

        def wb(m, _):
            r0 = sid * RPT + m * C
            pltpu.sync_copy(acc_sh.at[pl.ds(r0, C)], out_v)
            pltpu.sync_copy(out_v, acc_hbm.at[cid, pl.ds(r0, C)])
            return 0
        lax.fori_loop(0, NFULL, wb, 0)
        if REM:
            r0 = sid * RPT + NFULL * C
            pltpu.sync_copy(acc_sh.at[pl.ds(r0, REM)], out_v.at[pl.ds(0, REM)])
            pltpu.sync_copy(out_v.at[pl.ds(0, REM)],
                            acc_hbm.at[cid, pl.ds(r0, REM)])
        @pl.when(sid == 0)
        def _():
            pltpu.sync_copy(acc_sh.at[pl.ds(_NS * RPT, TAIL)],
                            out_v.at[pl.ds(0, TAIL)])
            pltpu.sync_copy(out_v.at[pl.ds(0, TAIL)],
                            acc_hbm.at[cid, pl.ds(_NS * RPT, TAIL)])

    return sc_kernel(dst3, src3, qn, kn, vn, kv)


def _sc_den_pass(dst3d, exr, N):
    NW, NCH, C = dst3d.shape
    E = NW * NCH * C
    EPW = E // NW
    RPT = (N // _NS) // 8 * 8
    TAIL = N - RPT * _NS
    NFULL, REM = RPT // C, RPT % C
    assert REM % 8 == 0 and TAIL <= C and TAIL % 8 == 0

    mesh = plsc.VectorSubcoreMesh(core_axis_name="c", subcore_axis_name="s",
                                  num_cores=_NC, num_subcores=_NS)

    @functools.partial(
        pl.kernel,
        out_type=jax.ShapeDtypeStruct((_NC, N, D), jnp.float32),
        mesh=mesh,
        scratch_types=[
            pltpu.VMEM((C,), jnp.int32),
            pltpu.VMEM((C, _LANES), jnp.float32),
            pltpu.VMEM((C, D), jnp.float32),
            pltpu.VMEM_SHARED((N, D), jnp.float32),
            pltpu.SemaphoreType.DMA,
        ],
    )
    def sc_kernel(dst_hbm, exr_hbm, den_hbm, dst_v, ex_v, row_v, den_sh, sem):
        cid = lax.axis_index("c")
        sid = lax.axis_index("s")
        wid = sid * _NC + cid

        def zrow(i, _):
            z = jnp.zeros((_LANES,), jnp.float32)
            for j in range(D // _LANES):
                row_v[i, _LANES * j:_LANES * (j + 1)] = z
            return 0
        lax.fori_loop(0, C, zrow, 0)

        def zcopy(m, _):
            pltpu.sync_copy(row_v, den_sh.at[pl.ds(sid * RPT + m * C, C)])
            return 0
        lax.fori_loop(0, NFULL, zcopy, 0)
        if REM:
            pltpu.sync_copy(row_v.at[pl.ds(0, REM)],
                            den_sh.at[pl.ds(sid * RPT + NFULL * C, REM)])
        @pl.when(sid == 0)
        def _():
            pltpu.sync_copy(row_v.at[pl.ds(0, TAIL)],
                            den_sh.at[pl.ds(_NS * RPT, TAIL)])
        plsc.subcore_barrier()

        def fill_body(i, _):
            row_v[i, 0:_LANES] = ex_v[i, :]
            return 0

        def chunk_body(j, _):
            base = wid * EPW + j * C
            pltpu.sync_copy(dst_hbm.at[wid, j], dst_v)
            pltpu.sync_copy(exr_hbm.at[pl.ds(base, C)], ex_v)
            lax.fori_loop(0, C, fill_body, 0)
            pltpu.sync_copy(row_v, den_sh.at[dst_v], add=True)
            return 0

        lax.fori_loop(0, NCH, chunk_body, 0)
        plsc.subcore_barrier()

        def wb(m, _):
            r0 = sid * RPT + m * C
            pltpu.sync_copy(den_sh.at[pl.ds(r0, C)], row_v)
            pltpu.sync_copy(row_v, den_hbm.at[cid, pl.ds(r0, C)])
            return 0
        lax.fori_loop(0, NFULL, wb, 0)
        if REM:
            r0 = sid * RPT + NFULL * C
            pltpu.sync_copy(den_sh.at[pl.ds(r0, REM)], row_v.at[pl.ds(0, REM)])
            pltpu.sync_copy(row_v.at[pl.ds(0, REM)],
                            den_hbm.at[cid, pl.ds(r0, REM)])
        @pl.when(sid == 0)
        def _():
            pltpu.sync_copy(den_sh.at[pl.ds(_NS * RPT, TAIL)],
                            row_v.at[pl.ds(0, TAIL)])
            pltpu.sync_copy(row_v.at[pl.ds(0, TAIL)],
                            den_hbm.at[cid, pl.ds(_NS * RPT, TAIL)])

    return sc_kernel(dst3d, exr)


def kernel(src_org_edge_feat, src_edge_to_time, src_center_node_idx,
           src_neigh_edge, src_node_features, params):
    p = params
    E = src_neigh_edge.shape[0]
    N = src_node_features.shape[0]
    dst = src_neigh_edge[:, 0]
    src = src_neigh_edge[:, 1]

    phi0 = jnp.cos(p["time_phase"])
    wef = jnp.concatenate([p[f"W{kv}{l}"][D:2 * D] for l in range(L)
                           for kv in ("k", "v")], axis=1)
    wt = jnp.concatenate([p[f"W{kv}{l}"][2 * D:3 * D] for l in range(L)
                          for kv in ("k", "v")], axis=1)
    projs = []
    for l in range(L):
        w = jnp.concatenate([p[f"Wq{l}"][:D], p[f"Wk{l}"][:D], p[f"Wv{l}"][:D]],
                            axis=1)
        q0 = phi0 @ p[f"Wq{l}"][D:]
        b = jnp.concatenate([q0, jnp.zeros((2 * D,), jnp.float32)])[None, :]
        projs.append((w, b))
    wa1 = p["Wa1"][:D] + p["Wa1"][D:]

    t2 = src_edge_to_time[:, None]
    tf2 = p["time_freq"][None, :]
    tp2 = p["time_phase"][None, :]

    NW = _NC * _NS
    dst3a = dst.reshape(NW, -1, 40)
    src3a = src.reshape(NW, -1, 40)
    dst3b = dst.reshape(NW, -1, 40)

    kvs = _edge_kv(src_org_edge_feat, t2, p["edge_W1"], p["edge_b1"][None, :],
                   p["edge_W2"], p["edge_b2"][None, :], wef, wt, tf2, tp2)
    h = _mlp2(src_node_features, p["node_W1"], p["node_b1"][None, :],
              p["node_W2"], p["node_b2"][None, :])

    for l in range(L):
        qkvn = _proj(h, projs[l][0], projs[l][1])
        qn, kn, vn = qkvn[:, :D], qkvn[:, D:2 * D], qkvn[:, 2 * D:]
        accp, exr = _sc_edge_pass(dst3a, src3a, qn, kn, vn, kvs[l])
        denp = _sc_den_pass(dst3b, exr, N)
        h = _upd(accp, denp, h, p[f"Wm1_{l}"][:D], p[f"Wm1_{l}"][D:],
                 p[f"bm1_{l}"][None, :], p[f"Wm2_{l}"], p[f"bm2_{l}"][None, :])

    x = h[src_center_node_idx]
    score = _mlp2(x, wa1, p["ba1"][None, :], p["Wa2"],
                  p["ba2"][None, :], blk=1024)
    return score
